# baseline (speedup 1.0000x reference)
import jax
import jax.numpy as jnp
from jax import lax
from jax.experimental import pallas as pl

S = 1024
A = 128
M = 512
N = 12
BB = 32


def _body(tok_ref, w_ref, sp_out, ab_out, it_out, mv_out):
    toks = tok_ref[...]
    sp3 = toks[:, :, 0:1]
    ab3 = toks[:, :, 1:2]
    it3 = toks[:, :, 2:3]
    mv3 = toks[:, :, 3:7]

    iota_s = lax.broadcasted_iota(jnp.int32, (N, BB, S), 2)
    iota_a = lax.broadcasted_iota(jnp.int32, (N, BB, A), 2)
    iota_m = lax.broadcasted_iota(jnp.int32, (N, BB, M), 2)

    oh3 = (iota_s == sp3 - 1).astype(jnp.float32)
    known3 = sp3 > 0
    counts = jnp.sum(oh3, axis=0)
    k2 = jnp.sum(known3.astype(jnp.float32), axis=0)
    inv_t2 = 1.0 / jnp.maximum(jnp.float32(S) - k2, 1.0)
    u2 = 1.0 - counts

    sp_out[...] = jnp.where(known3, oh3, u2 * inv_t2)

    e2 = jnp.where(known3, oh3, u2).astype(jnp.bfloat16).reshape(N * BB, S)
    g3 = jnp.dot(e2, w_ref[...],
                 preferred_element_type=jnp.float32).reshape(N, BB, 2 * A + M)
    r3 = g3 * jnp.where(known3, 1.0, inv_t2)
    r_ab = r3[:, :, 0:A]
    r_it = r3[:, :, A:2 * A]
    r_mv = r3[:, :, 2 * A:]

    am3 = ab3 > 0
    oh_ab = (iota_a == ab3 - 1).astype(jnp.float32)
    unk_a = r_ab / jnp.maximum(jnp.sum(r_ab, axis=2, keepdims=True), 1.0)
    ab_out[...] = jnp.where(am3, oh_ab, unk_a)

    oh_it = (iota_a == jnp.maximum(it3 - 1, 0)).astype(jnp.float32)
    unk_i = r_it / jnp.maximum(jnp.sum(r_it, axis=2, keepdims=True), 1.0)
    it_out[...] = jnp.where(am3, oh_it, unk_i)

    km = jnp.zeros((N, BB, M), jnp.float32)
    for j in range(4):
        km = km + (iota_m == mv3[:, :, j:j + 1] - 1).astype(jnp.float32)
    msum = jnp.sum(mv3, axis=2, keepdims=True)
    unk_m = r_mv - km
    unk_m = unk_m / jnp.maximum(jnp.sum(unk_m, axis=2, keepdims=True), 1.0)
    num_missing = 4.0 - jnp.sum((km > 0).astype(jnp.float32), axis=2,
                                keepdims=True)
    mm3 = known3 & (msum != 0)
    mv_out[...] = jnp.where(mm3, km + num_missing * unk_m, 4.0 * unk_m)


def kernel(species_token, ability_token, item_token, move_tokens,
           species_table, abilities_w, abilities_onehot,
           items_w, items_onehot, moves_w, moves_onehot):
    B = species_token.shape[0]
    D = 2 * A + M
    wcat = jnp.concatenate([abilities_w, items_w, moves_w],
                           axis=1).astype(jnp.bfloat16)

    tok_t = jnp.transpose(
        jnp.concatenate([species_token[:, :, None], ability_token[:, :, None],
                         item_token[:, :, None], move_tokens], axis=2),
        (1, 0, 2))

    outs = pl.pallas_call(
        _body,
        grid=(B // BB,),
        in_specs=[
            pl.BlockSpec((N, BB, 7), lambda i: (0, i, 0)),
            pl.BlockSpec((S, D), lambda i: (0, 0)),
        ],
        out_specs=[
            pl.BlockSpec((N, BB, S), lambda i: (0, i, 0)),
            pl.BlockSpec((N, BB, A), lambda i: (0, i, 0)),
            pl.BlockSpec((N, BB, A), lambda i: (0, i, 0)),
            pl.BlockSpec((N, BB, M), lambda i: (0, i, 0)),
        ],
        out_shape=[
            jax.ShapeDtypeStruct((N, B, S), jnp.float32),
            jax.ShapeDtypeStruct((N, B, A), jnp.float32),
            jax.ShapeDtypeStruct((N, B, A), jnp.float32),
            jax.ShapeDtypeStruct((N, B, M), jnp.float32),
        ],
    )(tok_t, wcat)

    return tuple(jnp.transpose(o, (1, 0, 2)) for o in outs)


# speedup vs baseline: 1.0616x; 1.0616x over previous
import jax
import jax.numpy as jnp
from jax import lax
from jax.experimental import pallas as pl

S = 1024
A = 128
M = 512
N = 12
BB = 32


def _body(tok_ref, wab_ref, wit_ref, wmv_ref, sp_out, ab_out, it_out, mv_out):
    toks = tok_ref[...]
    sp3 = toks[0:N]
    ab3 = toks[N:2 * N]
    it3 = toks[2 * N:3 * N]

    iota_s = lax.broadcasted_iota(jnp.int32, (N, BB, S), 2)
    iota_a = lax.broadcasted_iota(jnp.int32, (N, BB, A), 2)
    iota_m = lax.broadcasted_iota(jnp.int32, (N, BB, M), 2)

    oh3 = (iota_s == sp3 - 1).astype(jnp.float32)
    known3 = sp3 > 0
    counts = jnp.sum(oh3, axis=0)
    k2 = jnp.sum(known3.astype(jnp.float32), axis=0)
    inv_t2 = 1.0 / jnp.maximum(jnp.float32(S) - k2, 1.0)
    u2 = 1.0 - counts

    sp_out[...] = jnp.where(known3, oh3, u2 * inv_t2)

    e2 = jnp.where(known3, oh3, u2).astype(jnp.bfloat16).reshape(N * BB, S)
    scale3 = jnp.where(known3, 1.0, inv_t2)

    def unk(w_ref, width):
        g = jnp.dot(e2, w_ref[...].astype(jnp.bfloat16),
                    preferred_element_type=jnp.float32).reshape(N, BB, width)
        return g * scale3

    r_ab = unk(wab_ref, A)
    r_it = unk(wit_ref, A)
    r_mv = unk(wmv_ref, M)

    am3 = ab3 > 0
    oh_ab = (iota_a == ab3 - 1).astype(jnp.float32)
    unk_a = r_ab / jnp.maximum(jnp.sum(r_ab, axis=2, keepdims=True), 1.0)
    ab_out[...] = jnp.where(am3, oh_ab, unk_a)

    oh_it = (iota_a == jnp.maximum(it3 - 1, 0)).astype(jnp.float32)
    unk_i = r_it / jnp.maximum(jnp.sum(r_it, axis=2, keepdims=True), 1.0)
    it_out[...] = jnp.where(am3, oh_it, unk_i)

    km = jnp.zeros((N, BB, M), jnp.float32)
    msum = jnp.zeros((N, BB, 1), jnp.int32)
    for j in range(4):
        mvj = toks[(3 + j) * N:(4 + j) * N]
        km = km + (iota_m == mvj - 1).astype(jnp.float32)
        msum = msum + mvj
    unk_m = r_mv - km
    unk_m = unk_m / jnp.maximum(jnp.sum(unk_m, axis=2, keepdims=True), 1.0)
    num_missing = 4.0 - jnp.sum((km > 0).astype(jnp.float32), axis=2,
                                keepdims=True)
    mm3 = known3 & (msum != 0)
    mv_out[...] = jnp.where(mm3, km + num_missing * unk_m, 4.0 * unk_m)


def kernel(species_token, ability_token, item_token, move_tokens,
           species_table, abilities_w, abilities_onehot,
           items_w, items_onehot, moves_w, moves_onehot):
    B = species_token.shape[0]

    tok_t = jnp.concatenate(
        [species_token.T, ability_token.T, item_token.T]
        + [move_tokens[:, :, j].T for j in range(4)], axis=0)[:, :, None]

    outs = pl.pallas_call(
        _body,
        grid=(B // BB,),
        in_specs=[
            pl.BlockSpec((7 * N, BB, 1), lambda i: (0, i, 0)),
            pl.BlockSpec((S, A), lambda i: (0, 0)),
            pl.BlockSpec((S, A), lambda i: (0, 0)),
            pl.BlockSpec((S, M), lambda i: (0, 0)),
        ],
        out_specs=[
            pl.BlockSpec((N, BB, S), lambda i: (0, i, 0)),
            pl.BlockSpec((N, BB, A), lambda i: (0, i, 0)),
            pl.BlockSpec((N, BB, A), lambda i: (0, i, 0)),
            pl.BlockSpec((N, BB, M), lambda i: (0, i, 0)),
        ],
        out_shape=[
            jax.ShapeDtypeStruct((N, B, S), jnp.float32),
            jax.ShapeDtypeStruct((N, B, A), jnp.float32),
            jax.ShapeDtypeStruct((N, B, A), jnp.float32),
            jax.ShapeDtypeStruct((N, B, M), jnp.float32),
        ],
    )(tok_t, abilities_w, items_w, moves_w)

    return tuple(jnp.transpose(o, (1, 0, 2)) for o in outs)


# speedup vs baseline: 1.3595x; 1.2807x over previous
import jax
import jax.numpy as jnp
from jax import lax
from jax.experimental import pallas as pl
from jax.experimental.pallas import tpu as pltpu

S = 1024
A = 128
M = 512
N = 12
BB = 32


def _body(tok_ref, wab_ref, wit_ref, wmv_ref,
          sp_out, ab_out, it_out, mv_out, w16_ref):
    @pl.when(pl.program_id(0) == 0)
    def _():
        w16_ref[:, 0:A] = wab_ref[...].astype(jnp.bfloat16)
        w16_ref[:, A:2 * A] = wit_ref[...].astype(jnp.bfloat16)
        w16_ref[:, 2 * A:] = wmv_ref[...].astype(jnp.bfloat16)

    toks = tok_ref[...]

    def plane(c):
        return jnp.concatenate(
            [toks[:, c + n:c + n + 1].reshape(1, BB, 1) for n in range(N)],
            axis=0)

    sp3 = plane(0)
    ab3 = plane(N)
    it3 = plane(2 * N)

    iota_s = lax.broadcasted_iota(jnp.int32, (N, BB, S), 2)
    iota_a = lax.broadcasted_iota(jnp.int32, (N, BB, A), 2)
    iota_m = lax.broadcasted_iota(jnp.int32, (N, BB, M), 2)

    oh3 = (iota_s == sp3 - 1).astype(jnp.float32)
    known3 = sp3 > 0
    counts = jnp.sum(oh3, axis=0)
    k2 = jnp.sum(known3.astype(jnp.float32), axis=0)
    inv_t2 = 1.0 / jnp.maximum(jnp.float32(S) - k2, 1.0)
    u2 = 1.0 - counts

    sp_out[...] = jnp.where(known3, oh3, u2 * inv_t2)

    e2 = jnp.where(known3, oh3, u2).astype(jnp.bfloat16).reshape(N * BB, S)
    g3 = jnp.dot(e2, w16_ref[...],
                 preferred_element_type=jnp.float32).reshape(N, BB, 2 * A + M)
    r3 = g3 * jnp.where(known3, 1.0, inv_t2)
    r_ab = r3[:, :, 0:A]
    r_it = r3[:, :, A:2 * A]
    r_mv = r3[:, :, 2 * A:]

    am3 = ab3 > 0
    oh_ab = (iota_a == ab3 - 1).astype(jnp.float32)
    unk_a = r_ab / jnp.maximum(jnp.sum(r_ab, axis=2, keepdims=True), 1.0)
    ab_out[...] = jnp.where(am3, oh_ab, unk_a)

    oh_it = (iota_a == jnp.maximum(it3 - 1, 0)).astype(jnp.float32)
    unk_i = r_it / jnp.maximum(jnp.sum(r_it, axis=2, keepdims=True), 1.0)
    it_out[...] = jnp.where(am3, oh_it, unk_i)

    km = jnp.zeros((N, BB, M), jnp.float32)
    msum = jnp.zeros((N, BB, 1), jnp.int32)
    for j in range(4):
        mvj = plane((3 + j) * N)
        km = km + (iota_m == mvj - 1).astype(jnp.float32)
        msum = msum + mvj
    unk_m = r_mv - km
    unk_m = unk_m / jnp.maximum(jnp.sum(unk_m, axis=2, keepdims=True), 1.0)
    num_missing = 4.0 - jnp.sum((km > 0).astype(jnp.float32), axis=2,
                                keepdims=True)
    mm3 = known3 & (msum != 0)
    mv_out[...] = jnp.where(mm3, km + num_missing * unk_m, 4.0 * unk_m)


def kernel(species_token, ability_token, item_token, move_tokens,
           species_table, abilities_w, abilities_onehot,
           items_w, items_onehot, moves_w, moves_onehot):
    B = species_token.shape[0]

    tok_cat = jnp.concatenate(
        [species_token, ability_token, item_token]
        + [move_tokens[:, :, j] for j in range(4)], axis=1)

    outs = pl.pallas_call(
        _body,
        grid=(B // BB,),
        in_specs=[
            pl.BlockSpec((BB, 7 * N), lambda i: (i, 0)),
            pl.BlockSpec((S, A), lambda i: (0, 0)),
            pl.BlockSpec((S, A), lambda i: (0, 0)),
            pl.BlockSpec((S, M), lambda i: (0, 0)),
        ],
        out_specs=[
            pl.BlockSpec((N, BB, S), lambda i: (0, i, 0)),
            pl.BlockSpec((N, BB, A), lambda i: (0, i, 0)),
            pl.BlockSpec((N, BB, A), lambda i: (0, i, 0)),
            pl.BlockSpec((N, BB, M), lambda i: (0, i, 0)),
        ],
        out_shape=[
            jax.ShapeDtypeStruct((N, B, S), jnp.float32),
            jax.ShapeDtypeStruct((N, B, A), jnp.float32),
            jax.ShapeDtypeStruct((N, B, A), jnp.float32),
            jax.ShapeDtypeStruct((N, B, M), jnp.float32),
        ],
        scratch_shapes=[pltpu.VMEM((S, 2 * A + M), jnp.bfloat16)],
    )(tok_cat, abilities_w, items_w, moves_w)

    return tuple(jnp.transpose(o, (1, 0, 2)) for o in outs)
